# baseline (speedup 1.0000x reference)
import functools
import math

import jax
import jax.numpy as jnp
from jax import lax
from jax.experimental import pallas as pl
from jax.experimental.pallas import tpu as pltpu
from jax.experimental.pallas import tpu_sc as plsc

N = 10000
D = 128
E = 320000
BB = 100
MM = 100
NH = 16
DH = 8
DFF = 512

NC = 2
NS = 16
NW = NC * NS
CH = 128
NCHUNKS = E // CH
ITERS = -(-NCHUNKS // NW)
RCH = 16
NRCH = N // RCH
RITERS = -(-NRCH // NS)

@functools.lru_cache(maxsize=None)
def _sc_mesh():
    return plsc.VectorSubcoreMesh(
        core_axis_name="c", subcore_axis_name="s", num_cores=NC, num_subcores=NS
    )


def _sc_deg_body(ei_hbm, out_hbm, idxv, deg_local):
    c = lax.axis_index("c")
    s = lax.axis_index("s")
    wid = s * NC + c
    z16 = jnp.zeros((16,), jnp.float32)
    ones16 = jnp.ones((16,), jnp.float32)

    def zero_body(i, _):
        deg_local[pl.ds(i * 16, 16)] = z16
        return 0

    lax.fori_loop(0, N // 16, zero_body, 0)

    def chunk_body(j, _):
        cid = j * NW + wid

        @pl.when(cid < NCHUNKS)
        def _():
            pltpu.sync_copy(ei_hbm.at[1, pl.ds(cid * CH, CH)], idxv)
            for k in range(CH // 16):
                idx = idxv[pl.ds(k * 16, 16)]
                plsc.addupdate_scatter(deg_local, [idx], ones16)

        return 0

    lax.fori_loop(0, ITERS, chunk_body, 0)
    pltpu.sync_copy(deg_local, out_hbm.at[wid])


@functools.lru_cache(maxsize=None)
def _sc_deg_kernel():
    return pl.kernel(
        _sc_deg_body,
        out_type=jax.ShapeDtypeStruct((NW, N), jnp.float32),
        mesh=_sc_mesh(),
        scratch_types=[
            pltpu.VMEM((CH,), jnp.int32),
            pltpu.VMEM((N,), jnp.float32),
        ],
        compiler_params=pltpu.CompilerParams(needs_layout_passes=False),
    )


def _sc_deg(ei):
    return _sc_deg_kernel()(ei)


ITERS2 = 2 * (-(-NCHUNKS // (2 * NW)))
PITERS = ITERS2 // 2
EPAD = ITERS2 * NW * CH


def _sc_agg_body(g_hbm, eic_hbm, out_hbm, eibuf, rows, aggsh,
                 gsem0, gsem1, isem0, isem1):
    c = lax.axis_index("c")
    s = lax.axis_index("s")
    wid = s * NC + c
    z16 = jnp.zeros((16,), jnp.float32)
    gsems = (gsem0, gsem1)
    isems = (isem0, isem1)

    def zero_body(i, _):
        for jj in range(D // 16):
            rows[0, i, pl.ds(jj * 16, 16)] = z16
        return 0

    lax.fori_loop(0, RCH, zero_body, 0)

    def zero_chunk(t, _):
        cid = t * NS + s

        @pl.when(cid < NRCH)
        def _():
            pltpu.sync_copy(
                rows.at[0, pl.ds(0, RCH)], aggsh.at[pl.ds(cid * RCH, RCH)]
            )

        return 0

    lax.fori_loop(0, RITERS, zero_chunk, 0)
    plsc.subcore_barrier()

    def idx_issue(b, t):
        pltpu.async_copy(eic_hbm.at[wid, t], eibuf.at[b], isems[b])

    def idx_wait(b, t):
        pltpu.make_async_copy(eic_hbm.at[wid, t], eibuf.at[b], isems[b]).wait()

    def gather_launch(b):
        pltpu.async_copy(g_hbm.at[eibuf.at[b, 0]], rows.at[b], gsems[b])

    def gather_drain_scatter(b):
        pltpu.make_async_copy(
            g_hbm.at[eibuf.at[b, 0]], rows.at[b], gsems[b]
        ).wait()
        pltpu.sync_copy(rows.at[b], aggsh.at[eibuf.at[b, 1]], add=True)

    idx_issue(0, 0)
    idx_issue(1, 1)
    idx_wait(0, 0)
    gather_launch(0)

    def pair_body(g, _):
        t0 = 2 * g
        idx_wait(1, t0 + 1)
        gather_launch(1)
        gather_drain_scatter(0)

        @pl.when(g + 1 < PITERS)
        def _():
            idx_issue(0, t0 + 2)

        gather_drain_scatter(1)

        @pl.when(g + 1 < PITERS)
        def _():
            idx_wait(0, t0 + 2)
            gather_launch(0)
            idx_issue(1, t0 + 3)

        return 0

    lax.fori_loop(0, PITERS, pair_body, 0)
    plsc.subcore_barrier()

    def wb_chunk(t, _):
        cid = t * NS + s

        @pl.when(cid < NRCH)
        def _():
            base = cid * RCH
            pltpu.sync_copy(aggsh.at[pl.ds(base, RCH)], rows.at[0, pl.ds(0, RCH)])
            pltpu.sync_copy(
                rows.at[0, pl.ds(0, RCH)], out_hbm.at[c, pl.ds(base, RCH)]
            )

        return 0

    lax.fori_loop(0, RITERS, wb_chunk, 0)


@functools.lru_cache(maxsize=None)
def _sc_agg_kernel():
    return pl.kernel(
        _sc_agg_body,
        out_type=jax.ShapeDtypeStruct((NC, N, D), jnp.float32),
        mesh=_sc_mesh(),
        scratch_types=[
            pltpu.VMEM((2, 2, CH), jnp.int32),
            pltpu.VMEM((2, CH, D), jnp.float32),
            pltpu.VMEM_SHARED((N + 16, D), jnp.float32),
            pltpu.SemaphoreType.DMA,
            pltpu.SemaphoreType.DMA,
            pltpu.SemaphoreType.DMA,
            pltpu.SemaphoreType.DMA,
        ],
        compiler_params=pltpu.CompilerParams(needs_layout_passes=False),
    )


def _agg_edge_layout(ei):
    pad = EPAD - E
    src = jnp.concatenate([ei[0], jnp.zeros((pad,), jnp.int32)])
    dst = jnp.concatenate([ei[1], jnp.full((pad,), N, jnp.int32)])
    eic = jnp.stack(
        [src.reshape(ITERS2, NW, CH), dst.reshape(ITERS2, NW, CH)], axis=2
    ).transpose(1, 0, 2, 3)
    return eic


def _sc_agg(g, eic):
    return _sc_agg_kernel()(g, eic)


def _tc_g1_body(x_ref, w_ref, deg_ref, g_ref, dinv_ref):
    degsum = jnp.sum(deg_ref[...], axis=1, keepdims=True)
    dinv = lax.rsqrt(1.0 + degsum)
    h = jnp.dot(x_ref[...], w_ref[...], preferred_element_type=jnp.float32)
    g_ref[...] = h * dinv
    dinv_ref[...] = jnp.broadcast_to(dinv, dinv_ref.shape)


def _tc_g1(x_raw, W1, deg_t):
    blk = 1000
    grid = N // blk
    return pl.pallas_call(
        _tc_g1_body,
        grid=(grid,),
        in_specs=[
            pl.BlockSpec((blk, D), lambda i: (i, 0)),
            pl.BlockSpec((D, D), lambda i: (0, 0)),
            pl.BlockSpec((blk, NW), lambda i: (i, 0)),
        ],
        out_specs=[
            pl.BlockSpec((blk, D), lambda i: (i, 0)),
            pl.BlockSpec((blk, D), lambda i: (i, 0)),
        ],
        out_shape=[
            jax.ShapeDtypeStruct((N, D), jnp.float32),
            jax.ShapeDtypeStruct((N, D), jnp.float32),
        ],
    )(x_raw, W1, deg_t)


def _tc_mid_body(p_ref, g_ref, dinv_ref, b_ref, w_ref, out_ref):
    dinv = dinv_ref[...]
    x1 = dinv * (p_ref[0] + p_ref[1] + g_ref[...]) + b_ref[...][None, :]
    x1 = jnp.maximum(x1, 0.0)
    out_ref[...] = dinv * jnp.dot(
        x1, w_ref[...], preferred_element_type=jnp.float32
    )


def _tc_mid(p1, g1, dinvb, b1, W2):
    blk = 1000
    grid = N // blk
    return pl.pallas_call(
        _tc_mid_body,
        grid=(grid,),
        in_specs=[
            pl.BlockSpec((NC, blk, D), lambda i: (0, i, 0)),
            pl.BlockSpec((blk, D), lambda i: (i, 0)),
            pl.BlockSpec((blk, D), lambda i: (i, 0)),
            pl.BlockSpec((D,), lambda i: (0,)),
            pl.BlockSpec((D, D), lambda i: (0, 0)),
        ],
        out_specs=pl.BlockSpec((blk, D), lambda i: (i, 0)),
        out_shape=jax.ShapeDtypeStruct((N, D), jnp.float32),
    )(p1, g1, dinvb, b1, W2)


def _ln_rows(x, g, b):
    m = jnp.mean(x, axis=1, keepdims=True)
    xc = x - m
    v = jnp.mean(xc * xc, axis=1, keepdims=True)
    return xc * lax.rsqrt(v + 1e-5) * g[None, :] + b[None, :]


def _tc_tr_body(
    p_ref, g_ref, dinv_ref, b2_ref, enc_ref, hmask_ref,
    wq_ref, bq_ref, wk_ref, bk_ref, wv_ref, bv_ref, wo_ref, bo_ref,
    wff1_ref, bff1_ref, wff2_ref, bff2_ref,
    ln1g_ref, ln1b_ref, ln2g_ref, ln2b_ref,
    y_ref, kp_ref, vp_ref, att_ref,
):
    dinv = dinv_ref[0]
    x = dinv * (p_ref[0, 0] + p_ref[1, 0] + g_ref[0]) + b2_ref[...][None, :]
    x = jnp.maximum(x, 0.0)
    y = enc_ref[0]

    lane = lax.broadcasted_iota(jnp.int32, (MM, CH), 1)
    padm = jnp.where(lane < MM, 0.0, -1e30).astype(jnp.float32)
    scale = 1.0 / math.sqrt(DH)

    @pl.when(pl.program_id(0) == 0)
    def _():
        kp_ref[...] = jnp.zeros(kp_ref.shape, jnp.float32)
        vp_ref[...] = jnp.zeros(vp_ref.shape, jnp.float32)

    for l in range(2):
        q = jnp.dot(y, wq_ref[l], preferred_element_type=jnp.float32) + bq_ref[l][None, :]
        k = jnp.dot(x, wk_ref[l], preferred_element_type=jnp.float32) + bk_ref[l][None, :]
        v = jnp.dot(x, wv_ref[l], preferred_element_type=jnp.float32) + bv_ref[l][None, :]

        for h in range(NH):
            mask = hmask_ref[h][None, :]
            kp_ref[h * CH : h * CH + MM, :] = k * mask
            vp_ref[h * CH : h * CH + MM, :] = v * mask

        scores = lax.dot_general(
            q, kp_ref[...],
            dimension_numbers=(((1,), (1,)), ((), ())),
            preferred_element_type=jnp.float32,
        ) * scale

        for h in range(NH):
            sl = scores[:, h * CH : (h + 1) * CH] + padm
            mx = jnp.max(sl, axis=1, keepdims=True)
            ex = jnp.exp(sl - mx)
            att_ref[:, h * CH : (h + 1) * CH] = ex / jnp.sum(
                ex, axis=1, keepdims=True
            )

        o = jnp.dot(att_ref[...], vp_ref[...], preferred_element_type=jnp.float32)
        o = jnp.dot(o, wo_ref[l], preferred_element_type=jnp.float32) + bo_ref[l][None, :]
        y = _ln_rows(y + o, ln1g_ref[l], ln1b_ref[l])
        f = jnp.dot(y, wff1_ref[l], preferred_element_type=jnp.float32) + bff1_ref[l][None, :]
        f = jnp.maximum(f, 0.0)
        f = jnp.dot(f, wff2_ref[l], preferred_element_type=jnp.float32) + bff2_ref[l][None, :]
        y = _ln_rows(y + f, ln2g_ref[l], ln2b_ref[l])

    y_ref[0] = y


def _tc_transformer(p2, g2, dinvb, b2, enc, hmask, tw):
    Wq, bq, Wk, bk, Wv, bv, Wo, bo, Wff1, bff1, Wff2, bff2, ln1g, ln1b, ln2g, ln2b = tw
    p4 = p2.reshape(NC, BB, MM, D)
    g4 = g2.reshape(BB, MM, D)
    d4 = dinvb.reshape(BB, MM, D)
    full = lambda shape: pl.BlockSpec(shape, lambda i: tuple(0 for _ in shape))
    return pl.pallas_call(
        _tc_tr_body,
        grid=(BB,),
        in_specs=[
            pl.BlockSpec((NC, 1, MM, D), lambda i: (0, i, 0, 0)),
            pl.BlockSpec((1, MM, D), lambda i: (i, 0, 0)),
            pl.BlockSpec((1, MM, D), lambda i: (i, 0, 0)),
            full((D,)),
            pl.BlockSpec((1, MM, D), lambda i: (i, 0, 0)),
            full((NH, D)),
            full((2, D, D)), full((2, D)),
            full((2, D, D)), full((2, D)),
            full((2, D, D)), full((2, D)),
            full((2, D, D)), full((2, D)),
            full((2, D, DFF)), full((2, DFF)),
            full((2, DFF, D)), full((2, D)),
            full((2, D)), full((2, D)),
            full((2, D)), full((2, D)),
        ],
        out_specs=pl.BlockSpec((1, MM, D), lambda i: (i, 0, 0)),
        out_shape=jax.ShapeDtypeStruct((BB, MM, D), jnp.float32),
        scratch_shapes=[
            pltpu.VMEM((NH * CH, D), jnp.float32),
            pltpu.VMEM((NH * CH, D), jnp.float32),
            pltpu.VMEM((MM, NH * CH), jnp.float32),
        ],
    )(p4, g4, d4, b2, enc, hmask,
      Wq, bq, Wk, bk, Wv, bv, Wo, bo,
      Wff1, bff1, Wff2, bff2, ln1g, ln1b, ln2g, ln2b)


def kernel(enc_out_vari, x_enc, x_raw, edge_index, W1, b1, W2, b2,
           Wq, bq, Wk, bk, Wv, bv, Wo, bo, Wff1, bff1, Wff2, bff2,
           ln1_g, ln1_b, ln2_g, ln2_b):
    del x_enc
    ei = edge_index.astype(jnp.int32)

    deg_p = _sc_deg(ei)
    deg_t = deg_p.T

    eic = _agg_edge_layout(ei)
    g1, dinvb = _tc_g1(x_raw, W1, deg_t)
    p1 = _sc_agg(g1, eic)
    g2 = _tc_mid(p1, g1, dinvb, b1, W2)
    p2 = _sc_agg(g2, eic)

    hd = jnp.arange(D, dtype=jnp.int32) // DH
    hmask = (hd[None, :] == jnp.arange(NH, dtype=jnp.int32)[:, None]).astype(
        jnp.float32
    )
    tw = (Wq, bq, Wk, bk, Wv, bv, Wo, bo, Wff1, bff1, Wff2, bff2,
          ln1_g, ln1_b, ln2_g, ln2_b)
    return _tc_transformer(p2, g2, dinvb, b2, enc_out_vari, hmask, tw)


# speedup vs baseline: 1.7800x; 1.7800x over previous
import functools
import math

import jax
import jax.numpy as jnp
from jax import lax
from jax.experimental import pallas as pl
from jax.experimental.pallas import tpu as pltpu
from jax.experimental.pallas import tpu_sc as plsc

N = 10000
D = 128
E = 320000
BB = 100
MM = 100
NH = 16
DH = 8
DFF = 512

NC = 2
NS = 16
NW = NC * NS
CH = 128
NCHUNKS = E // CH
ITERS = -(-NCHUNKS // NW)
RCH = 16
NRCH = N // RCH
RITERS = -(-NRCH // NS)

@functools.lru_cache(maxsize=None)
def _sc_mesh():
    return plsc.VectorSubcoreMesh(
        core_axis_name="c", subcore_axis_name="s", num_cores=NC, num_subcores=NS
    )


def _sc_deg_body(ei_hbm, out_hbm, idxv, deg_local):
    c = lax.axis_index("c")
    s = lax.axis_index("s")
    wid = s * NC + c
    z16 = jnp.zeros((16,), jnp.float32)
    ones16 = jnp.ones((16,), jnp.float32)

    def zero_body(i, _):
        deg_local[pl.ds(i * 16, 16)] = z16
        return 0

    lax.fori_loop(0, N // 16, zero_body, 0)

    def chunk_body(j, _):
        cid = j * NW + wid

        @pl.when(cid < NCHUNKS)
        def _():
            pltpu.sync_copy(ei_hbm.at[1, pl.ds(cid * CH, CH)], idxv)
            for k in range(CH // 16):
                idx = idxv[pl.ds(k * 16, 16)]
                plsc.addupdate_scatter(deg_local, [idx], ones16)

        return 0

    lax.fori_loop(0, ITERS, chunk_body, 0)
    pltpu.sync_copy(deg_local, out_hbm.at[wid])


@functools.lru_cache(maxsize=None)
def _sc_deg_kernel():
    return pl.kernel(
        _sc_deg_body,
        out_type=jax.ShapeDtypeStruct((NW, N), jnp.float32),
        mesh=_sc_mesh(),
        scratch_types=[
            pltpu.VMEM((CH,), jnp.int32),
            pltpu.VMEM((N,), jnp.float32),
        ],
        compiler_params=pltpu.CompilerParams(needs_layout_passes=False),
    )


def _sc_deg(ei):
    return _sc_deg_kernel()(ei)


ITERS2 = 2 * (-(-NCHUNKS // (2 * NW)))
PITERS = ITERS2 // 2


def _sc_agg_body(g_hbm, ei_hbm, out_hbm, sidx, didx, rows, aggsh,
                 gsem0, gsem1, ssem0, ssem1, dsem0, dsem1):
    c = lax.axis_index("c")
    s = lax.axis_index("s")
    wid = s * NC + c
    z16 = jnp.zeros((16,), jnp.float32)
    junk16 = jnp.full((16,), N, jnp.int32)
    gsems = (gsem0, gsem1)
    ssems = (ssem0, ssem1)
    dsems = (dsem0, dsem1)

    def zero_body(i, _):
        for jj in range(D // 16):
            rows[0, i, pl.ds(jj * 16, 16)] = z16
        return 0

    lax.fori_loop(0, RCH, zero_body, 0)

    def zero_chunk(t, _):
        cid = t * NS + s

        @pl.when(cid < NRCH)
        def _():
            pltpu.sync_copy(
                rows.at[0, pl.ds(0, RCH)], aggsh.at[pl.ds(cid * RCH, RCH)]
            )

        return 0

    lax.fori_loop(0, RITERS, zero_chunk, 0)
    plsc.subcore_barrier()

    def idx_issue(b, t):
        cid = t * NW + wid

        @pl.when(cid < NCHUNKS)
        def _():
            pltpu.async_copy(ei_hbm.at[0, pl.ds(cid * CH, CH)], sidx.at[b],
                             ssems[b])
            pltpu.async_copy(ei_hbm.at[1, pl.ds(cid * CH, CH)], didx.at[b],
                             dsems[b])

        @pl.when(cid >= NCHUNKS)
        def _():
            for k in range(CH // 16):
                didx[b, pl.ds(k * 16, 16)] = junk16

    def idx_wait(b, t):
        cid = t * NW + wid

        @pl.when(cid < NCHUNKS)
        def _():
            pltpu.make_async_copy(ei_hbm.at[0, pl.ds(cid * CH, CH)],
                                  sidx.at[b], ssems[b]).wait()
            pltpu.make_async_copy(ei_hbm.at[1, pl.ds(cid * CH, CH)],
                                  didx.at[b], dsems[b]).wait()

    def gather_launch(b):
        pltpu.async_copy(g_hbm.at[sidx.at[b]], rows.at[b], gsems[b])

    def drain_scatter(b):
        pltpu.make_async_copy(g_hbm.at[sidx.at[b]], rows.at[b],
                              gsems[b]).wait()
        pltpu.sync_copy(rows.at[b], aggsh.at[didx.at[b]], add=True)

    idx_issue(0, 0)
    idx_issue(1, 1)
    idx_wait(0, 0)
    gather_launch(0)

    def pair_body(g, _):
        t0 = 2 * g
        idx_wait(1, t0 + 1)
        gather_launch(1)
        drain_scatter(0)

        @pl.when(g + 1 < PITERS)
        def _():
            idx_issue(0, t0 + 2)

        drain_scatter(1)

        @pl.when(g + 1 < PITERS)
        def _():
            idx_wait(0, t0 + 2)
            gather_launch(0)
            idx_issue(1, t0 + 3)

        return 0

    lax.fori_loop(0, PITERS, pair_body, 0)
    plsc.subcore_barrier()

    def wb_chunk(t, _):
        cid = t * NS + s

        @pl.when(cid < NRCH)
        def _():
            base = cid * RCH
            pltpu.sync_copy(aggsh.at[pl.ds(base, RCH)], rows.at[0, pl.ds(0, RCH)])
            pltpu.sync_copy(
                rows.at[0, pl.ds(0, RCH)], out_hbm.at[c, pl.ds(base, RCH)]
            )

        return 0

    lax.fori_loop(0, RITERS, wb_chunk, 0)


@functools.lru_cache(maxsize=None)
def _sc_agg_kernel():
    return pl.kernel(
        _sc_agg_body,
        out_type=jax.ShapeDtypeStruct((NC, N, D), jnp.float32),
        mesh=_sc_mesh(),
        scratch_types=[
            pltpu.VMEM((2, CH), jnp.int32),
            pltpu.VMEM((2, CH), jnp.int32),
            pltpu.VMEM((2, CH, D), jnp.float32),
            pltpu.VMEM_SHARED((N + 16, D), jnp.float32),
            pltpu.SemaphoreType.DMA,
            pltpu.SemaphoreType.DMA,
            pltpu.SemaphoreType.DMA,
            pltpu.SemaphoreType.DMA,
            pltpu.SemaphoreType.DMA,
            pltpu.SemaphoreType.DMA,
        ],
        compiler_params=pltpu.CompilerParams(needs_layout_passes=False),
    )


def _sc_agg(g, ei):
    return _sc_agg_kernel()(g, ei)


def _tc_g1_body(x_ref, w_ref, deg_ref, g_ref, dinv_ref):
    degsum = jnp.sum(deg_ref[...], axis=1, keepdims=True)
    dinv = lax.rsqrt(1.0 + degsum)
    h = jnp.dot(x_ref[...], w_ref[...], preferred_element_type=jnp.float32)
    g_ref[...] = h * dinv
    dinv_ref[...] = jnp.broadcast_to(dinv, dinv_ref.shape)


def _tc_g1(x_raw, W1, deg_t):
    blk = 1000
    grid = N // blk
    return pl.pallas_call(
        _tc_g1_body,
        grid=(grid,),
        in_specs=[
            pl.BlockSpec((blk, D), lambda i: (i, 0)),
            pl.BlockSpec((D, D), lambda i: (0, 0)),
            pl.BlockSpec((blk, NW), lambda i: (i, 0)),
        ],
        out_specs=[
            pl.BlockSpec((blk, D), lambda i: (i, 0)),
            pl.BlockSpec((blk, D), lambda i: (i, 0)),
        ],
        out_shape=[
            jax.ShapeDtypeStruct((N, D), jnp.float32),
            jax.ShapeDtypeStruct((N, D), jnp.float32),
        ],
    )(x_raw, W1, deg_t)


def _tc_mid_body(p_ref, g_ref, dinv_ref, b_ref, w_ref, out_ref):
    dinv = dinv_ref[...]
    x1 = dinv * (p_ref[0] + p_ref[1] + g_ref[...]) + b_ref[...][None, :]
    x1 = jnp.maximum(x1, 0.0)
    out_ref[...] = dinv * jnp.dot(
        x1, w_ref[...], preferred_element_type=jnp.float32
    )


def _tc_mid(p1, g1, dinvb, b1, W2):
    blk = 1000
    grid = N // blk
    return pl.pallas_call(
        _tc_mid_body,
        grid=(grid,),
        in_specs=[
            pl.BlockSpec((NC, blk, D), lambda i: (0, i, 0)),
            pl.BlockSpec((blk, D), lambda i: (i, 0)),
            pl.BlockSpec((blk, D), lambda i: (i, 0)),
            pl.BlockSpec((D,), lambda i: (0,)),
            pl.BlockSpec((D, D), lambda i: (0, 0)),
        ],
        out_specs=pl.BlockSpec((blk, D), lambda i: (i, 0)),
        out_shape=jax.ShapeDtypeStruct((N, D), jnp.float32),
    )(p1, g1, dinvb, b1, W2)


def _ln_rows(x, g, b):
    m = jnp.mean(x, axis=1, keepdims=True)
    xc = x - m
    v = jnp.mean(xc * xc, axis=1, keepdims=True)
    return xc * lax.rsqrt(v + 1e-5) * g[None, :] + b[None, :]


def _tc_tr_body(
    p_ref, g_ref, dinv_ref, b2_ref, enc_ref, hmask_ref,
    wq_ref, bq_ref, wk_ref, bk_ref, wv_ref, bv_ref, wo_ref, bo_ref,
    wff1_ref, bff1_ref, wff2_ref, bff2_ref,
    ln1g_ref, ln1b_ref, ln2g_ref, ln2b_ref,
    y_ref, kp_ref, vp_ref, att_ref,
):
    dinv = dinv_ref[0]
    x = dinv * (p_ref[0, 0] + p_ref[1, 0] + g_ref[0]) + b2_ref[...][None, :]
    x = jnp.maximum(x, 0.0)
    y = enc_ref[0]

    lane = lax.broadcasted_iota(jnp.int32, (MM, CH), 1)
    padm = jnp.where(lane < MM, 0.0, -1e30).astype(jnp.float32)
    scale = 1.0 / math.sqrt(DH)

    @pl.when(pl.program_id(0) == 0)
    def _():
        kp_ref[...] = jnp.zeros(kp_ref.shape, jnp.float32)
        vp_ref[...] = jnp.zeros(vp_ref.shape, jnp.float32)

    for l in range(2):
        q = jnp.dot(y, wq_ref[l], preferred_element_type=jnp.float32) + bq_ref[l][None, :]
        k = jnp.dot(x, wk_ref[l], preferred_element_type=jnp.float32) + bk_ref[l][None, :]
        v = jnp.dot(x, wv_ref[l], preferred_element_type=jnp.float32) + bv_ref[l][None, :]

        for h in range(NH):
            mask = hmask_ref[h][None, :]
            kp_ref[h * CH : h * CH + MM, :] = k * mask
            vp_ref[h * CH : h * CH + MM, :] = v * mask

        scores = lax.dot_general(
            q, kp_ref[...],
            dimension_numbers=(((1,), (1,)), ((), ())),
            preferred_element_type=jnp.float32,
        ) * scale

        for h in range(NH):
            sl = scores[:, h * CH : (h + 1) * CH] + padm
            mx = jnp.max(sl, axis=1, keepdims=True)
            ex = jnp.exp(sl - mx)
            att_ref[:, h * CH : (h + 1) * CH] = ex / jnp.sum(
                ex, axis=1, keepdims=True
            )

        o = jnp.dot(att_ref[...], vp_ref[...], preferred_element_type=jnp.float32)
        o = jnp.dot(o, wo_ref[l], preferred_element_type=jnp.float32) + bo_ref[l][None, :]
        y = _ln_rows(y + o, ln1g_ref[l], ln1b_ref[l])
        f = jnp.dot(y, wff1_ref[l], preferred_element_type=jnp.float32) + bff1_ref[l][None, :]
        f = jnp.maximum(f, 0.0)
        f = jnp.dot(f, wff2_ref[l], preferred_element_type=jnp.float32) + bff2_ref[l][None, :]
        y = _ln_rows(y + f, ln2g_ref[l], ln2b_ref[l])

    y_ref[0] = y


def _tc_transformer(p2, g2, dinvb, b2, enc, hmask, tw):
    Wq, bq, Wk, bk, Wv, bv, Wo, bo, Wff1, bff1, Wff2, bff2, ln1g, ln1b, ln2g, ln2b = tw
    p4 = p2.reshape(NC, BB, MM, D)
    g4 = g2.reshape(BB, MM, D)
    d4 = dinvb.reshape(BB, MM, D)
    full = lambda shape: pl.BlockSpec(shape, lambda i: tuple(0 for _ in shape))
    return pl.pallas_call(
        _tc_tr_body,
        grid=(BB,),
        in_specs=[
            pl.BlockSpec((NC, 1, MM, D), lambda i: (0, i, 0, 0)),
            pl.BlockSpec((1, MM, D), lambda i: (i, 0, 0)),
            pl.BlockSpec((1, MM, D), lambda i: (i, 0, 0)),
            full((D,)),
            pl.BlockSpec((1, MM, D), lambda i: (i, 0, 0)),
            full((NH, D)),
            full((2, D, D)), full((2, D)),
            full((2, D, D)), full((2, D)),
            full((2, D, D)), full((2, D)),
            full((2, D, D)), full((2, D)),
            full((2, D, DFF)), full((2, DFF)),
            full((2, DFF, D)), full((2, D)),
            full((2, D)), full((2, D)),
            full((2, D)), full((2, D)),
        ],
        out_specs=pl.BlockSpec((1, MM, D), lambda i: (i, 0, 0)),
        out_shape=jax.ShapeDtypeStruct((BB, MM, D), jnp.float32),
        scratch_shapes=[
            pltpu.VMEM((NH * CH, D), jnp.float32),
            pltpu.VMEM((NH * CH, D), jnp.float32),
            pltpu.VMEM((MM, NH * CH), jnp.float32),
        ],
    )(p4, g4, d4, b2, enc, hmask,
      Wq, bq, Wk, bk, Wv, bv, Wo, bo,
      Wff1, bff1, Wff2, bff2, ln1g, ln1b, ln2g, ln2b)


def kernel(enc_out_vari, x_enc, x_raw, edge_index, W1, b1, W2, b2,
           Wq, bq, Wk, bk, Wv, bv, Wo, bo, Wff1, bff1, Wff2, bff2,
           ln1_g, ln1_b, ln2_g, ln2_b):
    del x_enc
    ei = edge_index.astype(jnp.int32)

    deg_p = _sc_deg(ei)
    deg_t = deg_p.T

    g1, dinvb = _tc_g1(x_raw, W1, deg_t)
    p1 = _sc_agg(g1, ei)
    g2 = _tc_mid(p1, g1, dinvb, b1, W2)
    p2 = _sc_agg(g2, ei)

    hd = jnp.arange(D, dtype=jnp.int32) // DH
    hmask = (hd[None, :] == jnp.arange(NH, dtype=jnp.int32)[:, None]).astype(
        jnp.float32
    )
    tw = (Wq, bq, Wk, bk, Wv, bv, Wo, bo, Wff1, bff1, Wff2, bff2,
          ln1_g, ln1_b, ln2_g, ln2_b)
    return _tc_transformer(p2, g2, dinvb, b2, enc_out_vari, hmask, tw)


# speedup vs baseline: 1.8083x; 1.0159x over previous
import functools
import math

import jax
import jax.numpy as jnp
from jax import lax
from jax.experimental import pallas as pl
from jax.experimental.pallas import tpu as pltpu
from jax.experimental.pallas import tpu_sc as plsc

N = 10000
D = 128
E = 320000
BB = 100
MM = 100
NH = 16
DH = 8
DFF = 512

TRB = 2
NC = 2
NS = 16
NW = NC * NS
CH = 128
NCHUNKS = E // CH
ITERS = -(-NCHUNKS // NW)
RCH = 16
NRCH = N // RCH
RITERS = -(-NRCH // NS)

@functools.lru_cache(maxsize=None)
def _sc_mesh():
    return plsc.VectorSubcoreMesh(
        core_axis_name="c", subcore_axis_name="s", num_cores=NC, num_subcores=NS
    )


def _sc_deg_body(ei_hbm, out_hbm, idxv, deg_local):
    c = lax.axis_index("c")
    s = lax.axis_index("s")
    wid = s * NC + c
    z16 = jnp.zeros((16,), jnp.float32)
    ones16 = jnp.ones((16,), jnp.float32)

    def zero_body(i, _):
        deg_local[pl.ds(i * 16, 16)] = z16
        return 0

    lax.fori_loop(0, N // 16, zero_body, 0)

    def chunk_body(j, _):
        cid = j * NW + wid

        @pl.when(cid < NCHUNKS)
        def _():
            pltpu.sync_copy(ei_hbm.at[1, pl.ds(cid * CH, CH)], idxv)
            for k in range(CH // 16):
                idx = idxv[pl.ds(k * 16, 16)]
                plsc.addupdate_scatter(deg_local, [idx], ones16)

        return 0

    lax.fori_loop(0, ITERS, chunk_body, 0)
    pltpu.sync_copy(deg_local, out_hbm.at[wid])


@functools.lru_cache(maxsize=None)
def _sc_deg_kernel():
    return pl.kernel(
        _sc_deg_body,
        out_type=jax.ShapeDtypeStruct((NW, N), jnp.float32),
        mesh=_sc_mesh(),
        scratch_types=[
            pltpu.VMEM((CH,), jnp.int32),
            pltpu.VMEM((N,), jnp.float32),
        ],
        compiler_params=pltpu.CompilerParams(needs_layout_passes=False),
    )


def _sc_deg(ei):
    return _sc_deg_kernel()(ei)


ITERS2 = 2 * (-(-NCHUNKS // (2 * NW)))
PITERS = ITERS2 // 2


def _sc_agg_body(g_hbm, ei_hbm, out_hbm, sidx, didx, rows, aggsh,
                 gsem0, gsem1, ssem0, ssem1, dsem0, dsem1):
    c = lax.axis_index("c")
    s = lax.axis_index("s")
    wid = s * NC + c
    z16 = jnp.zeros((16,), jnp.float32)
    junk16 = jnp.full((16,), N, jnp.int32)
    gsems = (gsem0, gsem1)
    ssems = (ssem0, ssem1)
    dsems = (dsem0, dsem1)

    def zero_body(i, _):
        for jj in range(D // 16):
            rows[0, i, pl.ds(jj * 16, 16)] = z16
        return 0

    lax.fori_loop(0, RCH, zero_body, 0)

    def zero_chunk(t, _):
        cid = t * NS + s

        @pl.when(cid < NRCH)
        def _():
            pltpu.sync_copy(
                rows.at[0, pl.ds(0, RCH)], aggsh.at[pl.ds(cid * RCH, RCH)]
            )

        return 0

    lax.fori_loop(0, RITERS, zero_chunk, 0)
    plsc.subcore_barrier()

    def idx_issue(b, t):
        cid = t * NW + wid

        @pl.when(cid < NCHUNKS)
        def _():
            pltpu.async_copy(ei_hbm.at[0, pl.ds(cid * CH, CH)], sidx.at[b],
                             ssems[b])
            pltpu.async_copy(ei_hbm.at[1, pl.ds(cid * CH, CH)], didx.at[b],
                             dsems[b])

        @pl.when(cid >= NCHUNKS)
        def _():
            for k in range(CH // 16):
                didx[b, pl.ds(k * 16, 16)] = junk16

    def idx_wait(b, t):
        cid = t * NW + wid

        @pl.when(cid < NCHUNKS)
        def _():
            pltpu.make_async_copy(ei_hbm.at[0, pl.ds(cid * CH, CH)],
                                  sidx.at[b], ssems[b]).wait()
            pltpu.make_async_copy(ei_hbm.at[1, pl.ds(cid * CH, CH)],
                                  didx.at[b], dsems[b]).wait()

    def gather_launch(b):
        pltpu.async_copy(g_hbm.at[sidx.at[b]], rows.at[b], gsems[b])

    def drain_scatter(b):
        pltpu.make_async_copy(g_hbm.at[sidx.at[b]], rows.at[b],
                              gsems[b]).wait()
        pltpu.sync_copy(rows.at[b], aggsh.at[didx.at[b]], add=True)

    idx_issue(0, 0)
    idx_issue(1, 1)
    idx_wait(0, 0)
    gather_launch(0)

    def pair_body(g, _):
        t0 = 2 * g
        idx_wait(1, t0 + 1)
        gather_launch(1)
        drain_scatter(0)

        @pl.when(g + 1 < PITERS)
        def _():
            idx_issue(0, t0 + 2)

        drain_scatter(1)

        @pl.when(g + 1 < PITERS)
        def _():
            idx_wait(0, t0 + 2)
            gather_launch(0)
            idx_issue(1, t0 + 3)

        return 0

    lax.fori_loop(0, PITERS, pair_body, 0)
    plsc.subcore_barrier()

    def wb_chunk(t, _):
        cid = t * NS + s

        @pl.when(cid < NRCH)
        def _():
            base = cid * RCH
            pltpu.sync_copy(aggsh.at[pl.ds(base, RCH)], rows.at[0, pl.ds(0, RCH)])
            pltpu.sync_copy(
                rows.at[0, pl.ds(0, RCH)], out_hbm.at[c, pl.ds(base, RCH)]
            )

        return 0

    lax.fori_loop(0, RITERS, wb_chunk, 0)


@functools.lru_cache(maxsize=None)
def _sc_agg_kernel():
    return pl.kernel(
        _sc_agg_body,
        out_type=jax.ShapeDtypeStruct((NC, N, D), jnp.float32),
        mesh=_sc_mesh(),
        scratch_types=[
            pltpu.VMEM((2, CH), jnp.int32),
            pltpu.VMEM((2, CH), jnp.int32),
            pltpu.VMEM((2, CH, D), jnp.float32),
            pltpu.VMEM_SHARED((N + 16, D), jnp.float32),
            pltpu.SemaphoreType.DMA,
            pltpu.SemaphoreType.DMA,
            pltpu.SemaphoreType.DMA,
            pltpu.SemaphoreType.DMA,
            pltpu.SemaphoreType.DMA,
            pltpu.SemaphoreType.DMA,
        ],
        compiler_params=pltpu.CompilerParams(needs_layout_passes=False),
    )


def _sc_agg(g, ei):
    return _sc_agg_kernel()(g, ei)


def _tc_g1_body(x_ref, w_ref, deg_ref, g_ref, dinv_ref):
    degsum = jnp.sum(deg_ref[...], axis=1, keepdims=True)
    dinv = lax.rsqrt(1.0 + degsum)
    h = jnp.dot(x_ref[...], w_ref[...], preferred_element_type=jnp.float32)
    g_ref[...] = h * dinv
    dinv_ref[...] = jnp.broadcast_to(dinv, dinv_ref.shape)


def _tc_g1(x_raw, W1, deg_t):
    blk = 1000
    grid = N // blk
    return pl.pallas_call(
        _tc_g1_body,
        grid=(grid,),
        in_specs=[
            pl.BlockSpec((blk, D), lambda i: (i, 0)),
            pl.BlockSpec((D, D), lambda i: (0, 0)),
            pl.BlockSpec((blk, NW), lambda i: (i, 0)),
        ],
        out_specs=[
            pl.BlockSpec((blk, D), lambda i: (i, 0)),
            pl.BlockSpec((blk, D), lambda i: (i, 0)),
        ],
        out_shape=[
            jax.ShapeDtypeStruct((N, D), jnp.float32),
            jax.ShapeDtypeStruct((N, D), jnp.float32),
        ],
    )(x_raw, W1, deg_t)


def _tc_mid_body(p_ref, g_ref, dinv_ref, b_ref, w_ref, out_ref):
    dinv = dinv_ref[...]
    x1 = dinv * (p_ref[0] + p_ref[1] + g_ref[...]) + b_ref[...][None, :]
    x1 = jnp.maximum(x1, 0.0)
    out_ref[...] = dinv * jnp.dot(
        x1, w_ref[...], preferred_element_type=jnp.float32
    )


def _tc_mid(p1, g1, dinvb, b1, W2):
    blk = 1000
    grid = N // blk
    return pl.pallas_call(
        _tc_mid_body,
        grid=(grid,),
        in_specs=[
            pl.BlockSpec((NC, blk, D), lambda i: (0, i, 0)),
            pl.BlockSpec((blk, D), lambda i: (i, 0)),
            pl.BlockSpec((blk, D), lambda i: (i, 0)),
            pl.BlockSpec((D,), lambda i: (0,)),
            pl.BlockSpec((D, D), lambda i: (0, 0)),
        ],
        out_specs=pl.BlockSpec((blk, D), lambda i: (i, 0)),
        out_shape=jax.ShapeDtypeStruct((N, D), jnp.float32),
    )(p1, g1, dinvb, b1, W2)


def _ln_rows(x, g, b):
    m = jnp.mean(x, axis=1, keepdims=True)
    xc = x - m
    v = jnp.mean(xc * xc, axis=1, keepdims=True)
    return xc * lax.rsqrt(v + 1e-5) * g[None, :] + b[None, :]


def _tc_tr_body(
    p_ref, g_ref, dinv_ref, b2_ref, enc_ref, hmask_ref,
    wq_ref, bq_ref, wk_ref, bk_ref, wv_ref, bv_ref, wo_ref, bo_ref,
    wff1_ref, bff1_ref, wff2_ref, bff2_ref,
    ln1g_ref, ln1b_ref, ln2g_ref, ln2b_ref,
    y_ref,
):
    scale = 1.0 / math.sqrt(DH)

    for bb in range(TRB):
        _tc_tr_one(
            p_ref, g_ref, dinv_ref, b2_ref, enc_ref, hmask_ref,
            wq_ref, bq_ref, wk_ref, bk_ref, wv_ref, bv_ref, wo_ref, bo_ref,
            wff1_ref, bff1_ref, wff2_ref, bff2_ref,
            ln1g_ref, ln1b_ref, ln2g_ref, ln2b_ref,
            y_ref, bb, scale,
        )


def _tc_tr_one(
    p_ref, g_ref, dinv_ref, b2_ref, enc_ref, hmask_ref,
    wq_ref, bq_ref, wk_ref, bk_ref, wv_ref, bv_ref, wo_ref, bo_ref,
    wff1_ref, bff1_ref, wff2_ref, bff2_ref,
    ln1g_ref, ln1b_ref, ln2g_ref, ln2b_ref,
    y_ref, bb, scale,
):
    dinv = dinv_ref[bb]
    x = dinv * (p_ref[0, bb] + p_ref[1, bb] + g_ref[bb]) + b2_ref[...][None, :]
    x = jnp.maximum(x, 0.0)
    y = enc_ref[bb]

    for l in range(2):
        q = jnp.dot(y, wq_ref[l], preferred_element_type=jnp.float32) + bq_ref[l][None, :]
        q = q * scale
        k = jnp.dot(x, wk_ref[l], preferred_element_type=jnp.float32) + bk_ref[l][None, :]
        v = jnp.dot(x, wv_ref[l], preferred_element_type=jnp.float32) + bv_ref[l][None, :]

        o = None
        for h in range(NH):
            mask = hmask_ref[h][None, :]
            sl = lax.dot_general(
                q, k * mask,
                dimension_numbers=(((1,), (1,)), ((), ())),
                preferred_element_type=jnp.float32,
            )
            mx = jnp.max(sl, axis=1, keepdims=True)
            ex = jnp.exp(jnp.minimum(sl - mx, 0.0))
            att = ex / jnp.sum(ex, axis=1, keepdims=True)
            oh = jnp.dot(att, v * mask, preferred_element_type=jnp.float32)
            o = oh if o is None else o + oh

        o = jnp.dot(o, wo_ref[l], preferred_element_type=jnp.float32) + bo_ref[l][None, :]
        y = _ln_rows(y + o, ln1g_ref[l], ln1b_ref[l])
        f = jnp.dot(y, wff1_ref[l], preferred_element_type=jnp.float32) + bff1_ref[l][None, :]
        f = jnp.maximum(f, 0.0)
        f = jnp.dot(f, wff2_ref[l], preferred_element_type=jnp.float32) + bff2_ref[l][None, :]
        y = _ln_rows(y + f, ln2g_ref[l], ln2b_ref[l])

    y_ref[bb] = y


def _tc_transformer(p2, g2, dinvb, b2, enc, hmask, tw):
    Wq, bq, Wk, bk, Wv, bv, Wo, bo, Wff1, bff1, Wff2, bff2, ln1g, ln1b, ln2g, ln2b = tw
    p4 = p2.reshape(NC, BB, MM, D)
    g4 = g2.reshape(BB, MM, D)
    d4 = dinvb.reshape(BB, MM, D)
    full = lambda shape: pl.BlockSpec(shape, lambda i: tuple(0 for _ in shape))
    return pl.pallas_call(
        _tc_tr_body,
        grid=(BB // TRB,),
        in_specs=[
            pl.BlockSpec((NC, TRB, MM, D), lambda i: (0, i, 0, 0)),
            pl.BlockSpec((TRB, MM, D), lambda i: (i, 0, 0)),
            pl.BlockSpec((TRB, MM, D), lambda i: (i, 0, 0)),
            full((D,)),
            pl.BlockSpec((TRB, MM, D), lambda i: (i, 0, 0)),
            full((NH, D)),
            full((2, D, D)), full((2, D)),
            full((2, D, D)), full((2, D)),
            full((2, D, D)), full((2, D)),
            full((2, D, D)), full((2, D)),
            full((2, D, DFF)), full((2, DFF)),
            full((2, DFF, D)), full((2, D)),
            full((2, D)), full((2, D)),
            full((2, D)), full((2, D)),
        ],
        out_specs=pl.BlockSpec((TRB, MM, D), lambda i: (i, 0, 0)),
        out_shape=jax.ShapeDtypeStruct((BB, MM, D), jnp.float32),
    )(p4, g4, d4, b2, enc, hmask,
      Wq, bq, Wk, bk, Wv, bv, Wo, bo,
      Wff1, bff1, Wff2, bff2, ln1g, ln1b, ln2g, ln2b)


def kernel(enc_out_vari, x_enc, x_raw, edge_index, W1, b1, W2, b2,
           Wq, bq, Wk, bk, Wv, bv, Wo, bo, Wff1, bff1, Wff2, bff2,
           ln1_g, ln1_b, ln2_g, ln2_b):
    del x_enc
    ei = edge_index.astype(jnp.int32)

    deg_p = _sc_deg(ei)
    deg_t = deg_p.T

    g1, dinvb = _tc_g1(x_raw, W1, deg_t)
    p1 = _sc_agg(g1, ei)
    g2 = _tc_mid(p1, g1, dinvb, b1, W2)
    p2 = _sc_agg(g2, ei)

    hd = jnp.arange(D, dtype=jnp.int32) // DH
    hmask = (hd[None, :] == jnp.arange(NH, dtype=jnp.int32)[:, None]).astype(
        jnp.float32
    )
    tw = (Wq, bq, Wk, bk, Wv, bv, Wo, bo, Wff1, bff1, Wff2, bff2,
          ln1_g, ln1_b, ln2_g, ln2_b)
    return _tc_transformer(p2, g2, dinvb, b2, enc_out_vari, hmask, tw)


# speedup vs baseline: 1.8256x; 1.0096x over previous
import functools
import math

import jax
import jax.numpy as jnp
from jax import lax
from jax.experimental import pallas as pl
from jax.experimental.pallas import tpu as pltpu
from jax.experimental.pallas import tpu_sc as plsc

N = 10000
D = 128
E = 320000
BB = 100
MM = 100
NH = 16
DH = 8
DFF = 512

TRB = 2
NC = 2
NS = 16
NW = NC * NS
CH = 128
NCHUNKS = E // CH
ITERS = -(-NCHUNKS // NW)
RCH = 16
NRCH = N // RCH
RITERS = -(-NRCH // NS)

@functools.lru_cache(maxsize=None)
def _sc_mesh():
    return plsc.VectorSubcoreMesh(
        core_axis_name="c", subcore_axis_name="s", num_cores=NC, num_subcores=NS
    )


def _sc_deg_body(ei_hbm, out_hbm, idxv, deg_local):
    c = lax.axis_index("c")
    s = lax.axis_index("s")
    wid = s * NC + c
    z16 = jnp.zeros((16,), jnp.float32)
    ones16 = jnp.ones((16,), jnp.float32)

    def zero_body(i, _):
        deg_local[pl.ds(i * 16, 16)] = z16
        return 0

    lax.fori_loop(0, N // 16, zero_body, 0)

    def chunk_body(j, _):
        cid = j * NW + wid

        @pl.when(cid < NCHUNKS)
        def _():
            pltpu.sync_copy(ei_hbm.at[1, pl.ds(cid * CH, CH)], idxv)
            for k in range(CH // 16):
                idx = idxv[pl.ds(k * 16, 16)]
                plsc.addupdate_scatter(deg_local, [idx], ones16)

        return 0

    lax.fori_loop(0, ITERS, chunk_body, 0)
    pltpu.sync_copy(deg_local, out_hbm.at[wid])


@functools.lru_cache(maxsize=None)
def _sc_deg_kernel():
    return pl.kernel(
        _sc_deg_body,
        out_type=jax.ShapeDtypeStruct((NW, N), jnp.float32),
        mesh=_sc_mesh(),
        scratch_types=[
            pltpu.VMEM((CH,), jnp.int32),
            pltpu.VMEM((N,), jnp.float32),
        ],
        compiler_params=pltpu.CompilerParams(needs_layout_passes=False),
    )


def _sc_deg(ei):
    return _sc_deg_kernel()(ei)


ITERS2 = 2 * (-(-NCHUNKS // (2 * NW)))
PITERS = ITERS2 // 2


def _sc_agg_body(g_hbm, ei_hbm, out_hbm, sidx, didx, rows, aggsh,
                 gsem0, gsem1, ssem0, ssem1, dsem0, dsem1):
    c = lax.axis_index("c")
    s = lax.axis_index("s")
    wid = s * NC + c
    z16 = jnp.zeros((16,), jnp.float32)
    junk16 = jnp.full((16,), N, jnp.int32)
    gsems = (gsem0, gsem1)
    ssems = (ssem0, ssem1)
    dsems = (dsem0, dsem1)

    def zero_body(i, _):
        for jj in range(D // 16):
            rows[0, i, pl.ds(jj * 16, 16)] = z16
        return 0

    lax.fori_loop(0, RCH, zero_body, 0)

    def zero_chunk(t, _):
        cid = t * NS + s

        @pl.when(cid < NRCH)
        def _():
            pltpu.sync_copy(
                rows.at[0, pl.ds(0, RCH)], aggsh.at[pl.ds(cid * RCH, RCH)]
            )

        return 0

    lax.fori_loop(0, RITERS, zero_chunk, 0)
    plsc.subcore_barrier()

    def idx_issue(b, t):
        cid = t * NW + wid

        @pl.when(cid < NCHUNKS)
        def _():
            pltpu.async_copy(ei_hbm.at[0, pl.ds(cid * CH, CH)], sidx.at[b],
                             ssems[b])
            pltpu.async_copy(ei_hbm.at[1, pl.ds(cid * CH, CH)], didx.at[b],
                             dsems[b])

        @pl.when(cid >= NCHUNKS)
        def _():
            for k in range(CH // 16):
                didx[b, pl.ds(k * 16, 16)] = junk16

    def idx_wait(b, t):
        cid = t * NW + wid

        @pl.when(cid < NCHUNKS)
        def _():
            pltpu.make_async_copy(ei_hbm.at[0, pl.ds(cid * CH, CH)],
                                  sidx.at[b], ssems[b]).wait()
            pltpu.make_async_copy(ei_hbm.at[1, pl.ds(cid * CH, CH)],
                                  didx.at[b], dsems[b]).wait()

    def gather_launch(b):
        pltpu.async_copy(g_hbm.at[sidx.at[b]], rows.at[b], gsems[b])

    def drain_scatter(b):
        pltpu.make_async_copy(g_hbm.at[sidx.at[b]], rows.at[b],
                              gsems[b]).wait()
        pltpu.sync_copy(rows.at[b], aggsh.at[didx.at[b]], add=True)

    idx_issue(0, 0)
    idx_issue(1, 1)
    idx_wait(0, 0)
    gather_launch(0)

    def pair_body(g, _):
        t0 = 2 * g
        idx_wait(1, t0 + 1)
        gather_launch(1)
        drain_scatter(0)

        @pl.when(g + 1 < PITERS)
        def _():
            idx_issue(0, t0 + 2)

        drain_scatter(1)

        @pl.when(g + 1 < PITERS)
        def _():
            idx_wait(0, t0 + 2)
            gather_launch(0)
            idx_issue(1, t0 + 3)

        return 0

    lax.fori_loop(0, PITERS, pair_body, 0)
    plsc.subcore_barrier()

    def wb_chunk(t, _):
        cid = t * NS + s

        @pl.when(cid < NRCH)
        def _():
            base = cid * RCH
            pltpu.sync_copy(aggsh.at[pl.ds(base, RCH)], rows.at[0, pl.ds(0, RCH)])
            pltpu.sync_copy(
                rows.at[0, pl.ds(0, RCH)], out_hbm.at[c, pl.ds(base, RCH)]
            )

        return 0

    lax.fori_loop(0, RITERS, wb_chunk, 0)


@functools.lru_cache(maxsize=None)
def _sc_agg_kernel():
    return pl.kernel(
        _sc_agg_body,
        out_type=jax.ShapeDtypeStruct((NC, N, D), jnp.float32),
        mesh=_sc_mesh(),
        scratch_types=[
            pltpu.VMEM((2, CH), jnp.int32),
            pltpu.VMEM((2, CH), jnp.int32),
            pltpu.VMEM((2, CH, D), jnp.float32),
            pltpu.VMEM_SHARED((N + 16, D), jnp.float32),
            pltpu.SemaphoreType.DMA,
            pltpu.SemaphoreType.DMA,
            pltpu.SemaphoreType.DMA,
            pltpu.SemaphoreType.DMA,
            pltpu.SemaphoreType.DMA,
            pltpu.SemaphoreType.DMA,
        ],
        compiler_params=pltpu.CompilerParams(needs_layout_passes=False),
    )


def _sc_agg(g, ei):
    return _sc_agg_kernel()(g, ei)


def _tc_g1_body(x_ref, w_ref, deg_ref, g_ref, dinv_ref):
    degsum = jnp.sum(deg_ref[...], axis=0)
    dinv = lax.rsqrt(1.0 + degsum)[:, None]
    h = jnp.dot(x_ref[...], w_ref[...], preferred_element_type=jnp.float32)
    g_ref[...] = h * dinv
    dinv_ref[...] = jnp.broadcast_to(dinv, dinv_ref.shape)


def _tc_g1(x_raw, W1, deg_p):
    blk = 1024
    grid = -(-N // blk)
    return pl.pallas_call(
        _tc_g1_body,
        grid=(grid,),
        in_specs=[
            pl.BlockSpec((blk, D), lambda i: (i, 0)),
            pl.BlockSpec((D, D), lambda i: (0, 0)),
            pl.BlockSpec((NW, blk), lambda i: (0, i)),
        ],
        out_specs=[
            pl.BlockSpec((blk, D), lambda i: (i, 0)),
            pl.BlockSpec((blk, D), lambda i: (i, 0)),
        ],
        out_shape=[
            jax.ShapeDtypeStruct((N, D), jnp.float32),
            jax.ShapeDtypeStruct((N, D), jnp.float32),
        ],
    )(x_raw, W1, deg_p)


def _tc_mid_body(p_ref, g_ref, dinv_ref, b_ref, w_ref, out_ref):
    dinv = dinv_ref[...]
    x1 = dinv * (p_ref[0] + p_ref[1] + g_ref[...]) + b_ref[...][None, :]
    x1 = jnp.maximum(x1, 0.0)
    out_ref[...] = dinv * jnp.dot(
        x1, w_ref[...], preferred_element_type=jnp.float32
    )


def _tc_mid(p1, g1, dinvb, b1, W2):
    blk = 1000
    grid = N // blk
    return pl.pallas_call(
        _tc_mid_body,
        grid=(grid,),
        in_specs=[
            pl.BlockSpec((NC, blk, D), lambda i: (0, i, 0)),
            pl.BlockSpec((blk, D), lambda i: (i, 0)),
            pl.BlockSpec((blk, D), lambda i: (i, 0)),
            pl.BlockSpec((D,), lambda i: (0,)),
            pl.BlockSpec((D, D), lambda i: (0, 0)),
        ],
        out_specs=pl.BlockSpec((blk, D), lambda i: (i, 0)),
        out_shape=jax.ShapeDtypeStruct((N, D), jnp.float32),
    )(p1, g1, dinvb, b1, W2)


def _ln_rows(x, g, b):
    m = jnp.mean(x, axis=1, keepdims=True)
    xc = x - m
    v = jnp.mean(xc * xc, axis=1, keepdims=True)
    return xc * lax.rsqrt(v + 1e-5) * g[None, :] + b[None, :]


def _tc_tr_body(
    p_ref, g_ref, dinv_ref, b2_ref, enc_ref, hmask_ref,
    wq_ref, bq_ref, wk_ref, bk_ref, wv_ref, bv_ref, wo_ref, bo_ref,
    wff1_ref, bff1_ref, wff2_ref, bff2_ref,
    ln1g_ref, ln1b_ref, ln2g_ref, ln2b_ref,
    y_ref,
):
    scale = 1.0 / math.sqrt(DH)

    for bb in range(TRB):
        _tc_tr_one(
            p_ref, g_ref, dinv_ref, b2_ref, enc_ref, hmask_ref,
            wq_ref, bq_ref, wk_ref, bk_ref, wv_ref, bv_ref, wo_ref, bo_ref,
            wff1_ref, bff1_ref, wff2_ref, bff2_ref,
            ln1g_ref, ln1b_ref, ln2g_ref, ln2b_ref,
            y_ref, bb, scale,
        )


def _tc_tr_one(
    p_ref, g_ref, dinv_ref, b2_ref, enc_ref, hmask_ref,
    wq_ref, bq_ref, wk_ref, bk_ref, wv_ref, bv_ref, wo_ref, bo_ref,
    wff1_ref, bff1_ref, wff2_ref, bff2_ref,
    ln1g_ref, ln1b_ref, ln2g_ref, ln2b_ref,
    y_ref, bb, scale,
):
    dinv = dinv_ref[bb]
    x = dinv * (p_ref[0, bb] + p_ref[1, bb] + g_ref[bb]) + b2_ref[...][None, :]
    x = jnp.maximum(x, 0.0)
    y = enc_ref[bb]

    for l in range(2):
        q = jnp.dot(y, wq_ref[l], preferred_element_type=jnp.float32) + bq_ref[l][None, :]
        q = q * scale
        k = jnp.dot(x, wk_ref[l], preferred_element_type=jnp.float32) + bk_ref[l][None, :]
        v = jnp.dot(x, wv_ref[l], preferred_element_type=jnp.float32) + bv_ref[l][None, :]

        o = None
        for h in range(NH):
            mask = hmask_ref[h][None, :]
            sl = lax.dot_general(
                q, k * mask,
                dimension_numbers=(((1,), (1,)), ((), ())),
                preferred_element_type=jnp.float32,
            )
            mx = jnp.max(sl, axis=1, keepdims=True)
            ex = jnp.exp(jnp.minimum(sl - mx, 0.0))
            att = ex / jnp.sum(ex, axis=1, keepdims=True)
            oh = jnp.dot(att, v * mask, preferred_element_type=jnp.float32)
            o = oh if o is None else o + oh

        o = jnp.dot(o, wo_ref[l], preferred_element_type=jnp.float32) + bo_ref[l][None, :]
        y = _ln_rows(y + o, ln1g_ref[l], ln1b_ref[l])
        f = jnp.dot(y, wff1_ref[l], preferred_element_type=jnp.float32) + bff1_ref[l][None, :]
        f = jnp.maximum(f, 0.0)
        f = jnp.dot(f, wff2_ref[l], preferred_element_type=jnp.float32) + bff2_ref[l][None, :]
        y = _ln_rows(y + f, ln2g_ref[l], ln2b_ref[l])

    y_ref[bb] = y


def _tc_transformer(p2, g2, dinvb, b2, enc, hmask, tw):
    Wq, bq, Wk, bk, Wv, bv, Wo, bo, Wff1, bff1, Wff2, bff2, ln1g, ln1b, ln2g, ln2b = tw
    p4 = p2.reshape(NC, BB, MM, D)
    g4 = g2.reshape(BB, MM, D)
    d4 = dinvb.reshape(BB, MM, D)
    full = lambda shape: pl.BlockSpec(shape, lambda i: tuple(0 for _ in shape))
    return pl.pallas_call(
        _tc_tr_body,
        grid=(BB // TRB,),
        in_specs=[
            pl.BlockSpec((NC, TRB, MM, D), lambda i: (0, i, 0, 0)),
            pl.BlockSpec((TRB, MM, D), lambda i: (i, 0, 0)),
            pl.BlockSpec((TRB, MM, D), lambda i: (i, 0, 0)),
            full((D,)),
            pl.BlockSpec((TRB, MM, D), lambda i: (i, 0, 0)),
            full((NH, D)),
            full((2, D, D)), full((2, D)),
            full((2, D, D)), full((2, D)),
            full((2, D, D)), full((2, D)),
            full((2, D, D)), full((2, D)),
            full((2, D, DFF)), full((2, DFF)),
            full((2, DFF, D)), full((2, D)),
            full((2, D)), full((2, D)),
            full((2, D)), full((2, D)),
        ],
        out_specs=pl.BlockSpec((TRB, MM, D), lambda i: (i, 0, 0)),
        out_shape=jax.ShapeDtypeStruct((BB, MM, D), jnp.float32),
    )(p4, g4, d4, b2, enc, hmask,
      Wq, bq, Wk, bk, Wv, bv, Wo, bo,
      Wff1, bff1, Wff2, bff2, ln1g, ln1b, ln2g, ln2b)


def kernel(enc_out_vari, x_enc, x_raw, edge_index, W1, b1, W2, b2,
           Wq, bq, Wk, bk, Wv, bv, Wo, bo, Wff1, bff1, Wff2, bff2,
           ln1_g, ln1_b, ln2_g, ln2_b):
    del x_enc
    ei = edge_index.astype(jnp.int32)

    deg_p = _sc_deg(ei)

    g1, dinvb = _tc_g1(x_raw, W1, deg_p)
    p1 = _sc_agg(g1, ei)
    g2 = _tc_mid(p1, g1, dinvb, b1, W2)
    p2 = _sc_agg(g2, ei)

    hd = jnp.arange(D, dtype=jnp.int32) // DH
    hmask = (hd[None, :] == jnp.arange(NH, dtype=jnp.int32)[:, None]).astype(
        jnp.float32
    )
    tw = (Wq, bq, Wk, bk, Wv, bv, Wo, bo, Wff1, bff1, Wff2, bff2,
          ln1_g, ln1_b, ln2_g, ln2_b)
    return _tc_transformer(p2, g2, dinvb, b2, enc_out_vari, hmask, tw)
